# baseline (speedup 1.0000x reference)
import dataclasses
import functools

import jax
import jax.numpy as jnp
from jax import lax
from jax.experimental import pallas as pl
from jax.experimental.pallas import tpu as pltpu
from jax.experimental.pallas import tpu_sc as plsc

B = 8
L = 2000
F = 32000
D = 128
QW = 20
NE = 100000
NR = 500
NW = 40000
VERY_NEG = -100000000000.0

LP = 2048
FP = 32768
RP = 512
QP = 32
NC = 2
NS = 16
CH = 64
GCH = 128
KCH = FP // NS // CH
BPC = B // NC
BPP = 2

_mesh = plsc.VectorSubcoreMesh(core_axis_name="c", subcore_axis_name="s")

_sc_cp = pltpu.CompilerParams()
if "needs_layout_passes" in pltpu.CompilerParams.__dataclass_fields__:
    _sc_cp = dataclasses.replace(_sc_cp, needs_layout_passes=False)


@functools.partial(
    pl.kernel,
    mesh=_mesh,
    out_type=[
        jax.ShapeDtypeStruct((B * LP, D), jnp.float32),
        jax.ShapeDtypeStruct((B * QP, D), jnp.float32),
    ],
    scratch_types=[
        pltpu.VMEM((GCH,), jnp.int32),
        pltpu.VMEM((GCH, D), jnp.float32),
        pltpu.VMEM((8,), jnp.int32),
        pltpu.VMEM((8, D), jnp.float32),
        pltpu.SemaphoreType.DMA,
    ],
)
def _sc_gather(ent_tab, word_tab, le_idx, qt_idx, eg_out, qg_out,
               idx_v, row_v, idx8_v, row8_v, sem):
    w = lax.axis_index("s") * NC + lax.axis_index("c")

    @pl.loop(0, (B * LP) // (NC * NS) // GCH)
    def _ent(k):
        base = w * ((B * LP) // (NC * NS)) + k * GCH
        pltpu.sync_copy(le_idx.at[pl.ds(base, GCH)], idx_v)
        pltpu.async_copy(ent_tab.at[idx_v], row_v, sem).wait()
        pltpu.sync_copy(row_v, eg_out.at[pl.ds(base, GCH)])

    base8 = w * 8
    pltpu.sync_copy(qt_idx.at[pl.ds(base8, 8)], idx8_v)
    pltpu.async_copy(word_tab.at[idx8_v], row8_v, sem).wait()
    pltpu.sync_copy(row8_v, qg_out.at[pl.ds(base8, 8)])


@functools.partial(
    pl.kernel,
    mesh=_mesh,
    out_type=jax.ShapeDtypeStruct((B * LP, D), jnp.float32),
    compiler_params=_sc_cp,
    scratch_types=[
        pltpu.VMEM((BPP, KCH, CH), jnp.int32),
        pltpu.VMEM((BPP, KCH, CH), jnp.int32),
        pltpu.VMEM((BPP, KCH, CH), jnp.int32),
        pltpu.VMEM((CH, D), jnp.float32),
        pltpu.VMEM((CH, D), jnp.float32),
        pltpu.VMEM((CH, D), jnp.float32),
        pltpu.VMEM((CH, D), jnp.float32),
        pltpu.VMEM((RP,), jnp.float32),
        pltpu.VMEM((CH,), jnp.float32),
        pltpu.VMEM((CH, D), jnp.float32),
        pltpu.VMEM((CH, D), jnp.float32),
        pltpu.VMEM_SHARED((BPP * LP, D), jnp.float32),
        pltpu.SemaphoreType.DMA,
        pltpu.SemaphoreType.DMA,
        pltpu.SemaphoreType.DMA,
        pltpu.SemaphoreType.DMA,
    ],
)
def _sc_fact(ep, relself, pbc, hg_i, rr_i, tg_i, nb_out,
             hi_v, ri_v, ti_v, h0, h1, r0, r1, ptab, pch,
             m0, m1, acc, gs0, gs1, ss0, ss1):
    c = lax.axis_index("c")
    s = lax.axis_index("s")
    rows_per_sub = (BPP * LP) // NS
    bufs = ((h0, r0, m0, gs0, ss0), (h1, r1, m1, gs1, ss1))
    depth = len(bufs)

    @pl.loop(0, BPC // BPP)
    def _phase(ph):
        pltpu.sync_copy(hg_i.at[c, s, pl.ds(ph * BPP, BPP)], hi_v)
        pltpu.sync_copy(rr_i.at[c, s, pl.ds(ph * BPP, BPP)], ri_v)
        pltpu.sync_copy(tg_i.at[c, s, pl.ds(ph * BPP, BPP)], ti_v)

        @pl.loop(0, CH)
        def _zrow(r):
            for col in range(0, D, 16):
                m0[r, pl.ds(col, 16)] = jnp.zeros((16,), jnp.float32)

        for z in range(rows_per_sub // CH):
            pltpu.sync_copy(m0, acc.at[pl.ds(s * rows_per_sub + z * CH, CH)])

        plsc.subcore_barrier()

        def issue(bl2, k, hb, rb, gsem):
            return (pltpu.async_copy(ep.at[hi_v.at[bl2, k]], hb, gsem),
                    pltpu.async_copy(relself.at[ri_v.at[bl2, k]], rb, gsem))

        @pl.loop(0, BPP)
        def _batch(bl2):
            pltpu.sync_copy(pbc.at[c * BPC + ph * BPP + bl2], ptab)

            pend = {kk: issue(bl2, kk, bufs[kk][0], bufs[kk][1],
                              bufs[kk][3]) for kk in range(depth)}
            scat = {}
            for k in range(KCH):
                hb, rb, mb, gsem, ssem = bufs[k % depth]
                for cp in pend.pop(k):
                    cp.wait()
                if k >= depth:
                    scat.pop(k - depth).wait()

                for g in range(CH // 16):
                    r16 = ri_v.at[bl2, k][pl.ds(g * 16, 16)]
                    pch[pl.ds(g * 16, 16)] = plsc.load_gather(ptab, [r16])

                @pl.loop(0, CH, step=2)
                def _row(r):
                    for dr in range(2):
                        pv = plsc.load_gather(
                            pch, [jnp.full((16,), r + dr, jnp.int32)])
                        for col in range(0, D, 16):
                            slc = (r + dr, pl.ds(col, 16))
                            mb[slc] = (
                                jnp.maximum(hb[slc] + rb[slc], 0.0) * pv
                            )

                scat[k] = pltpu.async_copy(mb, acc.at[ti_v.at[bl2, k]],
                                           ssem, add=True)
                if k + depth < KCH:
                    pend[k + depth] = issue(bl2, k + depth, hb, rb, gsem)
            for k in range(KCH - depth, KCH):
                scat.pop(k).wait()

        plsc.subcore_barrier()

        @pl.loop(0, rows_per_sub // CH)
        def _flush(j):
            base = s * rows_per_sub + j * CH
            pltpu.sync_copy(acc.at[pl.ds(base, CH)], h0)
            pltpu.sync_copy(
                h0,
                nb_out.at[pl.ds(c * (BPC * LP) + ph * (BPP * LP) + base, CH)])


def _make_tc_prep():
    def body(relpad_ref, qg_ref, qt_ref, rlw_ref, rlb_ref, qw_ref, qb_ref,
             ksw_ref, ksb_ref,
             qvec_ref, rs0_ref, rs1_ref, pb0_ref, pb1_ref):
        qg = qg_ref[...].reshape(B, QP, D)
        m = (qt_ref[...] != NW).astype(jnp.float32)
        q_vec = (qg * m[:, :, None]).sum(axis=1) / jnp.clip(
            m.sum(axis=1, keepdims=True), 1.0, None)
        qvec_ref[...] = q_vec

        rel = jnp.dot(relpad_ref[...], rlw_ref[...]) + rlb_ref[...]
        rmask = (lax.broadcasted_iota(jnp.int32, (1, RP), 1) < NR
                 ).astype(jnp.float32)

        for i, (rs_ref, pb_ref) in enumerate(((rs0_ref, pb0_ref),
                                              (rs1_ref, pb1_ref))):
            rs_ref[...] = jnp.dot(rel, ksw_ref[i]) + ksb_ref[i]
            qi = jnp.dot(q_vec, qw_ref[i]) + qb_ref[i]
            dots = lax.dot_general(qi, rel, (((1,), (1,)), ((), ())))
            p = jax.nn.sigmoid(dots) * rmask
            pb_ref[...] = p

    return pl.pallas_call(
        body,
        out_shape=[
            jax.ShapeDtypeStruct((B, D), jnp.float32),
            jax.ShapeDtypeStruct((RP, D), jnp.float32),
            jax.ShapeDtypeStruct((RP, D), jnp.float32),
            jax.ShapeDtypeStruct((B, RP), jnp.float32),
            jax.ShapeDtypeStruct((B, RP), jnp.float32),
        ],
    )


def _make_tc_init():
    def body(eg_ref, q2e_ref, qvec_ref, elw_ref, elb_ref, ktw_ref, ktb_ref,
             ent_ref, ep_ref):
        ent = jnp.dot(eg_ref[...], elw_ref[...]) + elb_ref[...]
        ent = ent.reshape(B, LP, D) + q2e_ref[...][:, :, None] * \
            qvec_ref[...][:, None, :]
        ent = ent.reshape(B * LP, D)
        ent_ref[...] = ent
        ep_ref[...] = jnp.dot(ent, ktw_ref[...]) + ktb_ref[...]

    return pl.pallas_call(
        body,
        out_shape=[
            jax.ShapeDtypeStruct((B * LP, D), jnp.float32),
            jax.ShapeDtypeStruct((B * LP, D), jnp.float32),
        ],
    )


def _make_tc_layer():
    def body(ent_ref, nb_ref, ewa_ref, ewb_ref, eb_ref, ktw_ref, ktb_ref,
             ent2_ref, ep_ref):
        ent2 = jax.nn.relu(
            jnp.dot(ent_ref[...], ewa_ref[...])
            + jnp.dot(nb_ref[...], ewb_ref[...]) + eb_ref[...])
        ent2_ref[...] = ent2
        ep_ref[...] = jnp.dot(ent2, ktw_ref[...]) + ktb_ref[...]

    return pl.pallas_call(
        body,
        out_shape=[
            jax.ShapeDtypeStruct((B * LP, D), jnp.float32),
            jax.ShapeDtypeStruct((B * LP, D), jnp.float32),
        ],
    )


def _make_tc_final():
    def body(ent_ref, nb_ref, ewa_ref, ewb_ref, eb_ref, sw_ref, sb_ref,
             le_ref, score_ref):
        ent2 = jax.nn.relu(
            jnp.dot(ent_ref[...], ewa_ref[...])
            + jnp.dot(nb_ref[...], ewb_ref[...]) + eb_ref[...])
        sc = jnp.sum(ent2 * sw_ref[...][:, 0][None, :], axis=1)
        sc = sc.reshape(B, LP) + sb_ref[...]
        mask_pen = jnp.where(le_ref[...] == NE, VERY_NEG, 0.0)
        score_ref[...] = sc + mask_pen

    return pl.pallas_call(
        body,
        out_shape=jax.ShapeDtypeStruct((B, LP), jnp.float32),
    )


_tc_prep = _make_tc_prep()
_tc_init = _make_tc_init()
_tc_layer = _make_tc_layer()
_tc_final = _make_tc_final()


def _worker_layout(idx):
    a = idx.reshape(NC, BPC, NS, KCH, CH)
    return a.transpose(0, 2, 1, 3, 4)


def kernel(local_entity, q2e_adj_mat, fact_head, fact_tail, kb_fact_rel,
           query_text, answer_dist, entity_emb, relation_emb, word_emb,
           entity_linear_w, entity_linear_b, relation_linear_w,
           relation_linear_b, query_w, query_b, kb_self_w, kb_self_b,
           kb_tail_w, kb_tail_b, e2e_w, e2e_b, score_w, score_b):
    le = local_entity.astype(jnp.int32)
    le_pad = jnp.pad(le, ((0, 0), (0, LP - L)), constant_values=NE)
    q2e_pad = jnp.pad(q2e_adj_mat[:, :, 0], ((0, 0), (0, LP - L)))
    qt_pad = jnp.pad(query_text.astype(jnp.int32), ((0, 0), (0, QP - QW)),
                     constant_values=NW)
    fh = jnp.pad(fact_head.astype(jnp.int32), ((0, 0), (0, FP - F)))
    ft = jnp.pad(fact_tail.astype(jnp.int32), ((0, 0), (0, FP - F)))
    fr = jnp.pad(kb_fact_rel.astype(jnp.int32), ((0, 0), (0, FP - F)),
                 constant_values=NR)

    boff = jnp.arange(B, dtype=jnp.int32)[:, None]
    hg = _worker_layout(fh + boff * LP)
    rr = _worker_layout(fr)
    tg = _worker_layout(ft + (boff % BPP) * LP)

    relpad = jnp.pad(relation_emb, ((0, RP - (NR + 1)), (0, 0)))

    eg, qg = _sc_gather(entity_emb, word_emb, le_pad.reshape(-1),
                        qt_pad.reshape(-1))

    qvec, rs0, rs1, pb0, pb1 = _tc_prep(
        relpad, qg, qt_pad, relation_linear_w, relation_linear_b,
        query_w, query_b, kb_self_w, kb_self_b)

    ent0, ep0 = _tc_init(eg, q2e_pad, qvec, entity_linear_w, entity_linear_b,
                         kb_tail_w[0], kb_tail_b[0])

    nb0 = _sc_fact(ep0, rs0, pb0, hg, rr, tg)
    ent1, ep1 = _tc_layer(ent0, nb0, e2e_w[0, :D], e2e_w[0, D:], e2e_b[0],
                          kb_tail_w[1], kb_tail_b[1])

    nb1 = _sc_fact(ep1, rs1, pb1, hg, rr, tg)
    score = _tc_final(ent1, nb1, e2e_w[1, :D], e2e_w[1, D:], e2e_b[1],
                      score_w, score_b.reshape(1, 1), le_pad)

    return score[:, :L]


# speedup vs baseline: 1.2540x; 1.2540x over previous
import functools

import jax
import jax.numpy as jnp
from jax import lax
from jax.experimental import pallas as pl
from jax.experimental.pallas import tpu as pltpu
from jax.experimental.pallas import tpu_sc as plsc

B = 8
L = 2000
F = 32000
D = 128
QW = 20
NE = 100000
NR = 500
NW = 40000
VERY_NEG = -100000000000.0

LP = 2048
FP = 32768
RP = 512
QP = 32
NC = 2
NS = 16
CH = 64
GCH = 128
KCH = FP // NS // CH
BPC = B // NC
BPP = 2

_mesh = plsc.VectorSubcoreMesh(core_axis_name="c", subcore_axis_name="s")


@functools.partial(
    pl.kernel,
    mesh=_mesh,
    out_type=[
        jax.ShapeDtypeStruct((B * LP, D), jnp.float32),
        jax.ShapeDtypeStruct((B * QP, D), jnp.float32),
    ],
    scratch_types=[
        pltpu.VMEM((GCH,), jnp.int32),
        pltpu.VMEM((GCH, D), jnp.float32),
        pltpu.VMEM((8,), jnp.int32),
        pltpu.VMEM((8, D), jnp.float32),
        pltpu.SemaphoreType.DMA,
    ],
)
def _sc_gather(ent_tab, word_tab, le_idx, qt_idx, eg_out, qg_out,
               idx_v, row_v, idx8_v, row8_v, sem):
    w = lax.axis_index("s") * NC + lax.axis_index("c")

    @pl.loop(0, (B * LP) // (NC * NS) // GCH)
    def _ent(k):
        base = w * ((B * LP) // (NC * NS)) + k * GCH
        pltpu.sync_copy(le_idx.at[pl.ds(base, GCH)], idx_v)
        pltpu.async_copy(ent_tab.at[idx_v], row_v, sem).wait()
        pltpu.sync_copy(row_v, eg_out.at[pl.ds(base, GCH)])

    base8 = w * 8
    pltpu.sync_copy(qt_idx.at[pl.ds(base8, 8)], idx8_v)
    pltpu.async_copy(word_tab.at[idx8_v], row8_v, sem).wait()
    pltpu.sync_copy(row8_v, qg_out.at[pl.ds(base8, 8)])


@functools.partial(
    pl.kernel,
    mesh=_mesh,
    out_type=jax.ShapeDtypeStruct((B * LP, D), jnp.float32),
    scratch_types=[
        pltpu.VMEM((BPP, KCH, CH), jnp.int32),
        pltpu.VMEM((BPP, KCH, CH), jnp.int32),
        pltpu.VMEM((BPP, KCH, CH), jnp.int32),
        pltpu.VMEM((BPP, KCH, CH), jnp.int32),
        pltpu.VMEM((CH, D), jnp.float32),
        pltpu.VMEM((CH, D), jnp.float32),
        pltpu.VMEM((CH, D), jnp.float32),
        pltpu.VMEM((CH, D), jnp.float32),
        pltpu.VMEM((CH, D), jnp.float32),
        pltpu.VMEM((CH, D), jnp.float32),
        pltpu.VMEM((CH, D), jnp.float32),
        pltpu.VMEM((CH, D), jnp.float32),
        pltpu.VMEM_SHARED((BPP * LP, D), jnp.float32),
        pltpu.SemaphoreType.DMA,
        pltpu.SemaphoreType.DMA,
        pltpu.SemaphoreType.DMA,
        pltpu.SemaphoreType.DMA,
        pltpu.SemaphoreType.DMA,
    ],
)
def _sc_fact(ep, relself, pbc, hg_i, rr_i, pg_i, tg_i, nb_out,
             hi_v, ri_v, pi_v, ti_v, h0, h1, r0, r1, p0, p1,
             m0, m1, acc, gs0, gs1, ss0, ss1, xsem):
    c = lax.axis_index("c")
    s = lax.axis_index("s")
    rows_per_sub = (BPP * LP) // NS
    bufs = ((h0, r0, p0, m0, gs0, ss0), (h1, r1, p1, m1, gs1, ss1))
    depth = len(bufs)

    @pl.loop(0, BPC // BPP)
    def _phase(ph):
        pltpu.sync_copy(hg_i.at[c, s, pl.ds(ph * BPP, BPP)], hi_v)
        pltpu.sync_copy(rr_i.at[c, s, pl.ds(ph * BPP, BPP)], ri_v)
        pltpu.sync_copy(pg_i.at[c, s, pl.ds(ph * BPP, BPP)], pi_v)
        pltpu.sync_copy(tg_i.at[c, s, pl.ds(ph * BPP, BPP)], ti_v)

        @pl.loop(0, CH)
        def _zrow(r):
            for col in range(0, D, 16):
                m0[pl.ds(r, 1), pl.ds(col, 16)] = jnp.zeros((1, 16),
                                                            jnp.float32)

        for z in range(rows_per_sub // CH):
            pltpu.sync_copy(m0, acc.at[pl.ds(s * rows_per_sub + z * CH, CH)])

        plsc.subcore_barrier()

        def issue(bl2, k, hb, rb, pb2, gsem):
            return (pltpu.async_copy(ep.at[hi_v.at[bl2, k]], hb, gsem),
                    pltpu.async_copy(relself.at[ri_v.at[bl2, k]], rb, gsem),
                    pltpu.async_copy(pbc.at[pi_v.at[bl2, k]], pb2, gsem))

        @pl.loop(0, BPP)
        def _batch(bl2):
            pend = {kk: issue(bl2, kk, bufs[kk][0], bufs[kk][1], bufs[kk][2],
                              bufs[kk][4]) for kk in range(depth)}
            scat = {}
            for k in range(KCH):
                hb, rb, pb2, mb, gsem, ssem = bufs[k % depth]
                for cp in pend.pop(k):
                    cp.wait()
                if k >= depth:
                    scat.pop(k - depth).wait()

                @pl.loop(0, CH, step=2)
                def _row(r):
                    for dr in range(2):
                        pv = pb2[pl.ds(r + dr, 1), pl.ds(0, 16)]
                        for col in range(0, D, 16):
                            slc = (pl.ds(r + dr, 1), pl.ds(col, 16))
                            mb[slc] = (
                                jnp.maximum(hb[slc] + rb[slc], 0.0) * pv
                            )

                scat[k] = pltpu.async_copy(mb, acc.at[ti_v.at[bl2, k]],
                                           ssem, add=True)
                if k + depth < KCH:
                    pend[k + depth] = issue(bl2, k + depth, hb, rb, pb2, gsem)
            for k in range(KCH - depth, KCH):
                scat.pop(k).wait()

        plsc.subcore_barrier()

        @pl.loop(0, rows_per_sub // CH)
        def _flush(j):
            base = s * rows_per_sub + j * CH
            pltpu.sync_copy(acc.at[pl.ds(base, CH)], h0)
            pltpu.sync_copy(
                h0,
                nb_out.at[pl.ds(c * (BPC * LP) + ph * (BPP * LP) + base, CH)])


def _make_tc_prep():
    def body(relpad_ref, qg_ref, qt_ref, rlw_ref, rlb_ref, qw_ref, qb_ref,
             ksw_ref, ksb_ref,
             qvec_ref, rs0_ref, rs1_ref, pb0_ref, pb1_ref):
        qg = qg_ref[...].reshape(B, QP, D)
        m = (qt_ref[...] != NW).astype(jnp.float32)
        q_vec = (qg * m[:, :, None]).sum(axis=1) / jnp.clip(
            m.sum(axis=1, keepdims=True), 1.0, None)
        qvec_ref[...] = q_vec

        rel = jnp.dot(relpad_ref[...], rlw_ref[...]) + rlb_ref[...]
        rmask = (lax.broadcasted_iota(jnp.int32, (1, RP), 1) < NR
                 ).astype(jnp.float32)

        for i, (rs_ref, pb_ref) in enumerate(((rs0_ref, pb0_ref),
                                              (rs1_ref, pb1_ref))):
            rs_ref[...] = jnp.dot(rel, ksw_ref[i]) + ksb_ref[i]
            qi = jnp.dot(q_vec, qw_ref[i]) + qb_ref[i]
            dots = lax.dot_general(qi, rel, (((1,), (1,)), ((), ())))
            p = jax.nn.sigmoid(dots) * rmask
            pb_ref[...] = jnp.broadcast_to(p[:, :, None], (B, RP, D))

    return pl.pallas_call(
        body,
        out_shape=[
            jax.ShapeDtypeStruct((B, D), jnp.float32),
            jax.ShapeDtypeStruct((RP, D), jnp.float32),
            jax.ShapeDtypeStruct((RP, D), jnp.float32),
            jax.ShapeDtypeStruct((B, RP, D), jnp.float32),
            jax.ShapeDtypeStruct((B, RP, D), jnp.float32),
        ],
    )


def _make_tc_init():
    def body(eg_ref, q2e_ref, qvec_ref, elw_ref, elb_ref, ktw_ref, ktb_ref,
             ent_ref, ep_ref):
        ent = jnp.dot(eg_ref[...], elw_ref[...]) + elb_ref[...]
        ent = ent.reshape(B, LP, D) + q2e_ref[...][:, :, None] * \
            qvec_ref[...][:, None, :]
        ent = ent.reshape(B * LP, D)
        ent_ref[...] = ent
        ep_ref[...] = jnp.dot(ent, ktw_ref[...]) + ktb_ref[...]

    return pl.pallas_call(
        body,
        out_shape=[
            jax.ShapeDtypeStruct((B * LP, D), jnp.float32),
            jax.ShapeDtypeStruct((B * LP, D), jnp.float32),
        ],
    )


def _make_tc_layer():
    def body(ent_ref, nb_ref, ewa_ref, ewb_ref, eb_ref, ktw_ref, ktb_ref,
             ent2_ref, ep_ref):
        ent2 = jax.nn.relu(
            jnp.dot(ent_ref[...], ewa_ref[...])
            + jnp.dot(nb_ref[...], ewb_ref[...]) + eb_ref[...])
        ent2_ref[...] = ent2
        ep_ref[...] = jnp.dot(ent2, ktw_ref[...]) + ktb_ref[...]

    return pl.pallas_call(
        body,
        out_shape=[
            jax.ShapeDtypeStruct((B * LP, D), jnp.float32),
            jax.ShapeDtypeStruct((B * LP, D), jnp.float32),
        ],
    )


def _make_tc_final():
    def body(ent_ref, nb_ref, ewa_ref, ewb_ref, eb_ref, sw_ref, sb_ref,
             le_ref, score_ref):
        ent2 = jax.nn.relu(
            jnp.dot(ent_ref[...], ewa_ref[...])
            + jnp.dot(nb_ref[...], ewb_ref[...]) + eb_ref[...])
        sc = jnp.sum(ent2 * sw_ref[...][:, 0][None, :], axis=1)
        sc = sc.reshape(B, LP) + sb_ref[...]
        mask_pen = jnp.where(le_ref[...] == NE, VERY_NEG, 0.0)
        score_ref[...] = sc + mask_pen

    return pl.pallas_call(
        body,
        out_shape=jax.ShapeDtypeStruct((B, LP), jnp.float32),
    )


_tc_prep = _make_tc_prep()
_tc_init = _make_tc_init()
_tc_layer = _make_tc_layer()
_tc_final = _make_tc_final()


def _worker_layout(idx):
    a = idx.reshape(NC, BPC, NS, KCH, CH)
    return a.transpose(0, 2, 1, 3, 4)


def kernel(local_entity, q2e_adj_mat, fact_head, fact_tail, kb_fact_rel,
           query_text, answer_dist, entity_emb, relation_emb, word_emb,
           entity_linear_w, entity_linear_b, relation_linear_w,
           relation_linear_b, query_w, query_b, kb_self_w, kb_self_b,
           kb_tail_w, kb_tail_b, e2e_w, e2e_b, score_w, score_b):
    le = local_entity.astype(jnp.int32)
    le_pad = jnp.pad(le, ((0, 0), (0, LP - L)), constant_values=NE)
    q2e_pad = jnp.pad(q2e_adj_mat[:, :, 0], ((0, 0), (0, LP - L)))
    qt_pad = jnp.pad(query_text.astype(jnp.int32), ((0, 0), (0, QP - QW)),
                     constant_values=NW)
    fh = jnp.pad(fact_head.astype(jnp.int32), ((0, 0), (0, FP - F)))
    ft = jnp.pad(fact_tail.astype(jnp.int32), ((0, 0), (0, FP - F)))
    fr = jnp.pad(kb_fact_rel.astype(jnp.int32), ((0, 0), (0, FP - F)),
                 constant_values=NR)

    boff = jnp.arange(B, dtype=jnp.int32)[:, None]
    hg = _worker_layout(fh + boff * LP)
    rr = _worker_layout(fr)
    pg = _worker_layout(fr + boff * RP)
    tg = _worker_layout(ft + (boff % BPP) * LP)

    relpad = jnp.pad(relation_emb, ((0, RP - (NR + 1)), (0, 0)))

    eg, qg = _sc_gather(entity_emb, word_emb, le_pad.reshape(-1),
                        qt_pad.reshape(-1))

    qvec, rs0, rs1, pb0, pb1 = _tc_prep(
        relpad, qg, qt_pad, relation_linear_w, relation_linear_b,
        query_w, query_b, kb_self_w, kb_self_b)

    ent0, ep0 = _tc_init(eg, q2e_pad, qvec, entity_linear_w, entity_linear_b,
                         kb_tail_w[0], kb_tail_b[0])

    nb0 = _sc_fact(ep0, rs0, pb0.reshape(B * RP, D), hg, rr, pg, tg)
    ent1, ep1 = _tc_layer(ent0, nb0, e2e_w[0, :D], e2e_w[0, D:], e2e_b[0],
                          kb_tail_w[1], kb_tail_b[1])

    nb1 = _sc_fact(ep1, rs1, pb1.reshape(B * RP, D), hg, rr, pg, tg)
    score = _tc_final(ent1, nb1, e2e_w[1, :D], e2e_w[1, D:], e2e_b[1],
                      score_w, score_b.reshape(1, 1), le_pad)

    return score[:, :L]
